# baseline (speedup 1.0000x reference)
import functools

import numpy as np

import jax
import jax.numpy as jnp
from jax import lax
from jax.experimental import pallas as pl
from jax.experimental.pallas import tpu as pltpu
from jax.experimental.pallas import tpu_sc as plsc

NC = 2
NS = 16
NW = NC * NS

D = 256
NG = D // 16
NNZ = 32
CH = 8
HALF = CH // 2
GN = HALF * NNZ


def _worker_id():
  return lax.axis_index("s") * NC + lax.axis_index("c")


def _sc_feature_transform(white_flat, black_flat, stm_flat, ft_w, ft_b):
  B = stm_flat.shape[0]
  rw = B // NW
  nchunk = rw // CH

  mesh = plsc.VectorSubcoreMesh(core_axis_name="c", subcore_axis_name="s",
                                num_cores=NC, num_subcores=NS)

  @functools.partial(
      pl.kernel,
      out_type=jax.ShapeDtypeStruct((B, 2 * D), jnp.float32),
      mesh=mesh,
      scratch_types=[
          [pltpu.VMEM((GN,), jnp.int32)] * 4,
          pltpu.VMEM((rw + 8,), jnp.float32),
          pltpu.VMEM((D,), jnp.float32),
          pltpu.VMEM((GN, D // 2), jnp.int32),
          pltpu.VMEM((GN, D // 2), jnp.int32),
          pltpu.VMEM((2 * CH, D), jnp.float32),
          pltpu.VMEM((CH, 2 * D), jnp.float32),
          [pltpu.SemaphoreType.DMA] * 4,
          [pltpu.SemaphoreType.DMA] * 2,
          pltpu.SemaphoreType.DMA,
      ],
  )
  def ft_kernel(w_hbm, b_hbm, stm_hbm, table_hbm, ftb_hbm, out_hbm,
                idx_v, stm_v, ftb_v, rows0_v, rows1_v, acc_v, out_v,
                isems, gsems, osem):
    wid = _worker_id()
    row_base = wid * rw

    pltpu.sync_copy(stm_hbm.at[pl.ds(row_base, rw)], stm_v.at[pl.ds(0, rw)])
    pltpu.sync_copy(ftb_hbm, ftb_v)

    rows_bufs = (rows0_v, rows1_v)

    def idx_off(c, p):
      return row_base * NNZ + c * (CH * NNZ) + (p % 2) * GN

    def idx_src(p):
      return w_hbm if p < 2 else b_hbm

    def load_idx(c, p):
      return pltpu.async_copy(
          idx_src(p).at[pl.ds(idx_off(c, p), GN)], idx_v[p], isems[p])

    def gather(p):
      return pltpu.async_copy(
          table_hbm.at[idx_v[p]], rows_bufs[p % 2], gsems[p % 2])

    handles = [load_idx(0, p) for p in range(4)]
    handles[0].wait()
    gather(0)

    @pl.loop(0, nchunk)
    def _chunk(c):
      for p in range(4):
        nt = (p + 1) % 4
        nc = c if p < 3 else c + 1

        def _issue(_nt=nt, _nc=nc):
          pltpu.make_async_copy(
              idx_src(_nt).at[pl.ds(idx_off(_nc, _nt), GN)], idx_v[_nt],
              isems[_nt]).wait()
          gather(_nt)

        if p < 3:
          _issue()
        else:
          pl.when(c + 1 < nchunk)(_issue)

        pltpu.make_async_copy(
            table_hbm.at[idx_v[p]], rows_bufs[p % 2], gsems[p % 2]).wait()

        @pl.when(c + 1 < nchunk)
        def _(_p=p):
          load_idx(c + 1, _p)

        rows_v = rows_bufs[p % 2]
        arow = (p % 2) * HALF + (p // 2) * CH
        W = D // 2
        for r in range(HALF):
          for hp in range(2):
            base = r * NNZ * W + hp * (W // 2)
            @pl.loop(
                base, base + NNZ * W,
                step=W,
                init_carry=tuple(
                    ftb_v[pl.ds((hp * 4 + t % 4) * 16 + (t // 4) * 128, 16)]
                    for t in range(8)),
                unroll=4)
            def _reduce(off, accs, _rows=rows_v):
              new = list(accs)
              for t in range(4):
                wv = _rows[0, pl.ds(off + t * 16, 16)]
                a = lax.bitcast_convert_type(
                    jnp.left_shift(wv, 16), jnp.float32)
                b = lax.bitcast_convert_type(
                    jnp.bitwise_and(wv, jnp.int32(-65536)), jnp.float32)
                new[t] = new[t] + a
                new[t + 4] = new[t + 4] + b
              return tuple(new)
            for t in range(8):
              acc_v[arow + r,
                    pl.ds((hp * 4 + t % 4) * 16 + (t // 4) * 128, 16)] = (
                        _reduce[t])

      @pl.when(c > 0)
      def _():
        pltpu.make_async_copy(
            out_v, out_hbm.at[pl.ds(row_base + (c - 1) * CH, CH)],
            osem).wait()

      sv = stm_v[pl.ds(c * CH, 16)]
      for r in range(CH):
        s = sv[r]
        for g in range(2 * NG):
          half = g // NG
          gg = g % NG
          w = acc_v[r, pl.ds(gg * 16, 16)]
          b = acc_v[CH + r, pl.ds(gg * 16, 16)]
          w = jnp.minimum(jnp.maximum(w, 0.0), 1.0)
          b = jnp.minimum(jnp.maximum(b, 0.0), 1.0)
          first = w if half == 0 else b
          second = b if half == 0 else w
          out_v[r, pl.ds(g * 16, 16)] = s * first + (1.0 - s) * second

      pltpu.async_copy(out_v, out_hbm.at[pl.ds(row_base + c * CH, CH)], osem)

    pltpu.make_async_copy(
        out_v, out_hbm.at[pl.ds(row_base + (nchunk - 1) * CH, CH)],
        osem).wait()

  return ft_kernel(white_flat, black_flat, stm_flat, ft_w, ft_b)


def _mlp_body(acc_ref, w1_ref, b1_ref, w2_ref, b2_ref, w3_ref, b3_ref,
              out_ref):
  x = acc_ref[...]
  h = jnp.dot(x, w1_ref[...], preferred_element_type=jnp.float32) + b1_ref[...]
  h = jnp.clip(h, 0.0, 1.0)
  h = jnp.dot(h, w2_ref[...], preferred_element_type=jnp.float32) + b2_ref[...]
  h = jnp.clip(h, 0.0, 1.0)
  out_ref[...] = (
      jnp.dot(h, w3_ref[...], preferred_element_type=jnp.float32) + b3_ref[...]
  )


def _mlp(acc, l1_w, l1_b, l2_w, l2_b, l3_w, l3_b):
  B = acc.shape[0]
  BM = min(2048, B)
  grid = (B // BM,)
  full = lambda shape: pl.BlockSpec(shape, lambda i: (0, 0))
  return pl.pallas_call(
      _mlp_body,
      grid=grid,
      in_specs=[
          pl.BlockSpec((BM, 2 * D), lambda i: (i, 0)),
          full((2 * D, 32)), full((1, 32)),
          full((32, 32)), full((1, 32)),
          full((32, 1)), full((1, 1)),
      ],
      out_specs=pl.BlockSpec((BM, 1), lambda i: (i, 0)),
      out_shape=jax.ShapeDtypeStruct((B, 1), jnp.float32),
  )(acc, l1_w.T, l1_b[None, :], l2_w.T, l2_b[None, :], l3_w.T,
    l3_b[None, :])


def kernel(white_features, black_features, stm, ft_w, ft_b, l1_w, l1_b,
           l2_w, l2_b, l3_w, l3_b):
  white_flat = white_features.reshape(-1).astype(jnp.int32)
  black_flat = black_features.reshape(-1).astype(jnp.int32)
  stm_flat = stm.reshape(-1)

  cols = np.arange(D).reshape(D // 32, 16, 2)
  perm = np.concatenate([cols[:, :, 0].ravel(), cols[:, :, 1].ravel()])
  ftb_perm = ft_b[perm]
  l1_wp = jnp.concatenate(
      [l1_w[:, :D][:, perm], l1_w[:, D:][:, perm]], axis=1)

  table_i32 = lax.bitcast_convert_type(
      ft_w.astype(jnp.bfloat16).reshape(ft_w.shape[0], D // 2, 2), jnp.int32)

  acc = _sc_feature_transform(white_flat, black_flat, stm_flat,
                              table_i32, ftb_perm)
  return _mlp(acc, l1_wp, l1_b, l2_w, l2_b, l3_w, l3_b)


# speedup vs baseline: 1.2541x; 1.2541x over previous
import functools

import jax
import jax.numpy as jnp
from jax import lax
from jax.experimental import pallas as pl
from jax.experimental.pallas import tpu as pltpu
from jax.experimental.pallas import tpu_sc as plsc

NC = 2
NS = 16
NW = NC * NS

D = 256
NG = D // 16
NNZ = 32
CH = 8
RPP = 2
NPH = CH // RPP * 2
NBUF = 4
DEPTH = 3
GN = RPP * NNZ


def _worker_id():
  return lax.axis_index("s") * NC + lax.axis_index("c")


def _sc_feature_transform(white_flat, black_flat, stm_flat, ft_w, ft_b):
  B = stm_flat.shape[0]
  rw = B // NW
  nchunk = rw // CH

  mesh = plsc.VectorSubcoreMesh(core_axis_name="c", subcore_axis_name="s",
                                num_cores=NC, num_subcores=NS)

  @functools.partial(
      pl.kernel,
      out_type=jax.ShapeDtypeStruct((B, 2 * D), jnp.float32),
      mesh=mesh,
      scratch_types=[
          [pltpu.VMEM((GN,), jnp.int32)] * NPH,
          pltpu.VMEM((rw + 8,), jnp.float32),
          pltpu.VMEM((D,), jnp.float32),
          [pltpu.VMEM((GN, D), jnp.float32)] * NBUF,
          pltpu.VMEM((2 * CH, D), jnp.float32),
          pltpu.VMEM((CH, 2 * D), jnp.float32),
          [pltpu.SemaphoreType.DMA] * NPH,
          [pltpu.SemaphoreType.DMA] * NBUF,
          pltpu.SemaphoreType.DMA,
      ],
  )
  def ft_kernel(w_hbm, b_hbm, stm_hbm, table_hbm, ftb_hbm, out_hbm,
                idx_v, stm_v, ftb_v, rows_bufs, acc_v, out_v,
                isems, gsems, osem):
    wid = _worker_id()
    row_base = wid * rw

    pltpu.sync_copy(stm_hbm.at[pl.ds(row_base, rw)], stm_v.at[pl.ds(0, rw)])
    pltpu.sync_copy(ftb_hbm, ftb_v)

    def idx_off(c, q):
      return row_base * NNZ + c * (CH * NNZ) + (q % (NPH // 2)) * GN

    def idx_src(q):
      return w_hbm if q < NPH // 2 else b_hbm

    def load_idx(c, q):
      return pltpu.async_copy(
          idx_src(q).at[pl.ds(idx_off(c, q), GN)], idx_v[q], isems[q])

    def gather(q):
      return pltpu.async_copy(
          table_hbm.at[idx_v[q]], rows_bufs[q % NBUF], gsems[q % NBUF])

    handles = [load_idx(0, q) for q in range(NPH)]
    for q in range(DEPTH):
      handles[q].wait()
      gather(q)

    @pl.loop(0, nchunk)
    def _chunk(c):
      for q in range(NPH):
        nt = (q + DEPTH) % NPH
        nc = c if q + DEPTH < NPH else c + 1

        def _issue(_nt=nt, _nc=nc):
          pltpu.make_async_copy(
              idx_src(_nt).at[pl.ds(idx_off(_nc, _nt), GN)], idx_v[_nt],
              isems[_nt]).wait()
          gather(_nt)

        if q + DEPTH < NPH:
          _issue()
        else:
          pl.when(c + 1 < nchunk)(_issue)

        pltpu.make_async_copy(
            table_hbm.at[idx_v[q]], rows_bufs[q % NBUF], gsems[q % NBUF],
        ).wait()

        @pl.when(c + 1 < nchunk)
        def _(_q=q):
          load_idx(c + 1, _q)

        rows_v = rows_bufs[q % NBUF]
        side = q // (NPH // 2)
        sub = q % (NPH // 2)
        for r in range(RPP):
          @pl.loop(
              0, NNZ,
              init_carry=tuple(
                  ftb_v[pl.ds(g * 16, 16)] for g in range(NG)),
              unroll=4)
          def _reduce(j, accs, _r=r, _rows=rows_v):
            row = _r * NNZ + j
            return tuple(
                accs[g] + _rows[row, pl.ds(g * 16, 16)] for g in range(NG))
          arow = side * CH + sub * RPP + r
          for g in range(NG):
            acc_v[arow, pl.ds(g * 16, 16)] = _reduce[g]

      @pl.when(c > 0)
      def _():
        pltpu.make_async_copy(
            out_v, out_hbm.at[pl.ds(row_base + (c - 1) * CH, CH)],
            osem).wait()

      sv = stm_v[pl.ds(c * CH, 16)]
      for r in range(CH):
        s = sv[r]
        for g in range(2 * NG):
          half = g // NG
          gg = g % NG
          w = acc_v[r, pl.ds(gg * 16, 16)]
          b = acc_v[CH + r, pl.ds(gg * 16, 16)]
          w = jnp.minimum(jnp.maximum(w, 0.0), 1.0)
          b = jnp.minimum(jnp.maximum(b, 0.0), 1.0)
          first = w if half == 0 else b
          second = b if half == 0 else w
          out_v[r, pl.ds(g * 16, 16)] = s * first + (1.0 - s) * second

      pltpu.async_copy(out_v, out_hbm.at[pl.ds(row_base + c * CH, CH)], osem)

    pltpu.make_async_copy(
        out_v, out_hbm.at[pl.ds(row_base + (nchunk - 1) * CH, CH)],
        osem).wait()

  return ft_kernel(white_flat, black_flat, stm_flat, ft_w, ft_b)


def _mlp_body(acc_ref, w1_ref, b1_ref, w2_ref, b2_ref, w3_ref, b3_ref,
              out_ref):
  x = acc_ref[...]
  h = jnp.dot(x, w1_ref[...], preferred_element_type=jnp.float32) + b1_ref[...]
  h = jnp.clip(h, 0.0, 1.0)
  h = jnp.dot(h, w2_ref[...], preferred_element_type=jnp.float32) + b2_ref[...]
  h = jnp.clip(h, 0.0, 1.0)
  out_ref[...] = (
      jnp.dot(h, w3_ref[...], preferred_element_type=jnp.float32) + b3_ref[...]
  )


def _mlp(acc, l1_w, l1_b, l2_w, l2_b, l3_w, l3_b):
  B = acc.shape[0]
  BM = min(2048, B)
  grid = (B // BM,)
  full = lambda shape: pl.BlockSpec(shape, lambda i: (0, 0))
  return pl.pallas_call(
      _mlp_body,
      grid=grid,
      in_specs=[
          pl.BlockSpec((BM, 2 * D), lambda i: (i, 0)),
          full((2 * D, 32)), full((1, 32)),
          full((32, 32)), full((1, 32)),
          full((32, 1)), full((1, 1)),
      ],
      out_specs=pl.BlockSpec((BM, 1), lambda i: (i, 0)),
      out_shape=jax.ShapeDtypeStruct((B, 1), jnp.float32),
  )(acc, l1_w.T, l1_b[None, :], l2_w.T, l2_b[None, :], l3_w.T,
    l3_b[None, :])


def kernel(white_features, black_features, stm, ft_w, ft_b, l1_w, l1_b,
           l2_w, l2_b, l3_w, l3_b):
  white_flat = white_features.reshape(-1).astype(jnp.int32)
  black_flat = black_features.reshape(-1).astype(jnp.int32)
  stm_flat = stm.reshape(-1)

  acc = _sc_feature_transform(white_flat, black_flat, stm_flat, ft_w, ft_b)
  return _mlp(acc, l1_w, l1_b, l2_w, l2_b, l3_w, l3_b)
